# baseline (speedup 1.0000x reference)
import jax
import jax.numpy as jnp
from jax.experimental import pallas as pl
from jax.experimental.pallas import tpu as pltpu

_P = 1024
_D = 1024
_TEMP = 0.01
_K = 103
_BLK = 128


def _body(x_ref, V_ref, alpha_ref, out_ref):
    logits = alpha_ref[:, :] * (1.0 / _TEMP)
    m = jnp.max(logits, axis=0, keepdims=True)
    e = jnp.exp(logits - m)
    s = jnp.sum(e, axis=0, keepdims=True)
    aw = jnp.clip(e * (_K / s), 0.0, 1.0)

    x16 = x_ref[:, :].astype(jnp.bfloat16)
    col = jax.lax.broadcasted_iota(jnp.int32, (_P, _BLK), 1)

    acc = None
    for k in range(_D // _BLK):
        blk = slice(k * _BLK, (k + 1) * _BLK)
        A = (V_ref[:, blk] * aw).astype(jnp.bfloat16)
        if k:
            A = jnp.roll(A, k * _BLK, axis=0)
        for b in range(7):
            shift = 1 << b
            A = jnp.where((col & shift) != 0, jnp.roll(A, shift, axis=0), A)
        part = jax.lax.dot_general(
            x16[:, blk], A, (((1,), (1,)), ((), ())),
            preferred_element_type=jnp.float32)
        acc = part if acc is None else acc + part

    out_ref[:, :] = acc


@jax.jit
def kernel(x, V, alpha):
    B = x.shape[0]
    return pl.pallas_call(
        _body,
        out_shape=jax.ShapeDtypeStruct((B, _P), x.dtype),
    )(x, V, alpha.reshape(_P, 1))


# speedup vs baseline: 1.0178x; 1.0178x over previous
import jax
import jax.numpy as jnp
from jax.experimental import pallas as pl
from jax.experimental.pallas import tpu as pltpu

_P = 1024
_D = 1024
_TEMP = 0.01
_K = 103
_BLK = 512


def _body(x_ref, V_ref, alpha_ref, out_ref):
    logits = alpha_ref[:, :] * (1.0 / _TEMP)
    m = jnp.max(logits, axis=0, keepdims=True)
    e = jnp.exp(logits - m)
    s = jnp.sum(e, axis=0, keepdims=True)
    aw = jnp.clip(e * (_K / s), 0.0, 1.0)

    x16 = x_ref[:, :].astype(jnp.bfloat16)
    col = jax.lax.broadcasted_iota(jnp.int32, (_P, _BLK), 1)

    acc = None
    for k in range(_D // _BLK):
        blk = slice(k * _BLK, (k + 1) * _BLK)
        A = (V_ref[:, blk] * aw).astype(jnp.bfloat16)
        if k:
            A = jnp.roll(A, k * _BLK, axis=0)
        for b in range(9):
            shift = 1 << b
            A = jnp.where((col & shift) != 0, jnp.roll(A, shift, axis=0), A)
        part = jax.lax.dot_general(
            x16[:, blk], A, (((1,), (1,)), ((), ())),
            preferred_element_type=jnp.float32)
        acc = part if acc is None else acc + part

    out_ref[:, :] = acc


@jax.jit
def kernel(x, V, alpha):
    B = x.shape[0]
    return pl.pallas_call(
        _body,
        out_shape=jax.ShapeDtypeStruct((B, _P), x.dtype),
    )(x, V, alpha.reshape(_P, 1))


# speedup vs baseline: 1.0352x; 1.0171x over previous
import jax
import jax.numpy as jnp
from jax.experimental import pallas as pl
from jax.experimental.pallas import tpu as pltpu

_P = 1024
_D = 1024
_TEMP = 0.01
_K = 103
_BLK = 256


def _body(x_ref, V_ref, alpha_ref, out_ref):
    logits = alpha_ref[:, :] * (1.0 / _TEMP)
    m = jnp.max(logits, axis=0, keepdims=True)
    e = jnp.exp(logits - m)
    s = jnp.sum(e, axis=0, keepdims=True)
    aw = jnp.clip(e * (_K / s), 0.0, 1.0)

    x16 = x_ref[:, :].astype(jnp.bfloat16)
    col = jax.lax.broadcasted_iota(jnp.int32, (_P, _BLK), 1)

    acc = None
    for k in range(_D // _BLK):
        blk = slice(k * _BLK, (k + 1) * _BLK)
        A = (V_ref[:, blk] * aw).astype(jnp.bfloat16)
        if k:
            A = jnp.roll(A, k * _BLK, axis=0)
        for b in range(8):
            shift = 1 << b
            A = jnp.where((col & shift) != 0, jnp.roll(A, shift, axis=0), A)
        part = jax.lax.dot_general(
            x16[:, blk], A, (((1,), (1,)), ((), ())),
            preferred_element_type=jnp.float32)
        acc = part if acc is None else acc + part

    out_ref[:, :] = acc


@jax.jit
def kernel(x, V, alpha):
    B = x.shape[0]
    return pl.pallas_call(
        _body,
        out_shape=jax.ShapeDtypeStruct((B, _P), x.dtype),
    )(x, V, alpha.reshape(_P, 1))


# speedup vs baseline: 1.0425x; 1.0071x over previous
import jax
import jax.numpy as jnp
from jax.experimental import pallas as pl

_P = 1024
_D = 1024
_TEMP = 0.01
_K = 103
_BLK = 256


def _body(x_ref, V_ref, alpha_ref, out_ref):
    logits = alpha_ref[:, :] * (1.0 / _TEMP)
    m = jnp.max(logits, axis=0, keepdims=True)
    e = jnp.exp(logits - m)
    s = jnp.sum(e, axis=0, keepdims=True)
    aw = jnp.clip(e * (_K / s), 0.0, 1.0)

    x16 = x_ref[:, :].astype(jnp.bfloat16)
    col = jax.lax.broadcasted_iota(jnp.int32, (_P, _BLK), 1)
    masks = [(col & (1 << b)) != 0 for b in range(8)]

    acc = None
    for k in range(_D // _BLK):
        blk = slice(k * _BLK, (k + 1) * _BLK)
        A = (V_ref[:, blk] * aw).astype(jnp.bfloat16)
        if k:
            A = jnp.roll(A, k * _BLK, axis=0)
        for b in range(8):
            A = jnp.where(masks[b], jnp.roll(A, 1 << b, axis=0), A)
        part = jax.lax.dot_general(
            x16[:, blk], A, (((1,), (1,)), ((), ())),
            preferred_element_type=jnp.float32)
        acc = part if acc is None else acc + part

    out_ref[:, :] = acc


@jax.jit
def kernel(x, V, alpha):
    B = x.shape[0]
    return pl.pallas_call(
        _body,
        out_shape=jax.ShapeDtypeStruct((B, _P), x.dtype),
    )(x, V, alpha.reshape(_P, 1))
